# baseline (speedup 1.0000x reference)
import functools

import jax
import jax.numpy as jnp
from jax import lax
from jax.experimental import pallas as pl
from jax.experimental.pallas import tpu as pltpu
from jax.experimental.pallas import tpu_sc as plsc

R, C = 64, 32768
L = 16
NC, NS = 2, 16
NW = NC * NS
ROWS_PER_W = R // NW
NCHUNK = C // L
UNROLL = 8
NITER = NCHUNK // UNROLL

ZBLK = 4096
PBLK = 512


def _row_argmax(row_v):
    lanes = lax.iota(jnp.int32, L)

    def body(i, carry):
        vmaxs, vits, iv = carry
        base = i * (UNROLL * L)
        new_maxs, new_its = [], []
        for k in range(UNROLL):
            v = row_v[pl.ds(base + k * L, L)]
            m = v > vmaxs[k]
            new_maxs.append(jnp.where(m, v, vmaxs[k]))
            new_its.append(jnp.where(m, iv, vits[k]))
        return tuple(new_maxs), tuple(new_its), iv + 1

    vmax0 = tuple(jnp.full((L,), -jnp.inf, jnp.float32) for _ in range(UNROLL))
    vit0 = tuple(jnp.zeros((L,), jnp.int32) for _ in range(UNROLL))
    iv0 = jnp.zeros((L,), jnp.int32)
    vmaxs, vits, _ = lax.fori_loop(0, NITER, body, (vmax0, vit0, iv0))

    vmaxs = list(vmaxs)
    vflat = [(vits[k] * UNROLL + k) * L + lanes for k in range(UNROLL)]

    n = UNROLL
    while n > 1:
        h = n // 2
        for k in range(h):
            a_m, a_i = vmaxs[k], vflat[k]
            b_m, b_i = vmaxs[k + h], vflat[k + h]
            better = (b_m > a_m) | ((b_m == a_m) & (b_i < a_i))
            vmaxs[k] = jnp.where(better, b_m, a_m)
            vflat[k] = jnp.where(better, b_i, a_i)
        n = h
    vmax, vidx = vmaxs[0], vflat[0]

    bm, bi = vmax[0], vidx[0]
    for i in range(1, L):
        m, idx = vmax[i], vidx[i]
        better = (m > bm) | ((m == bm) & (idx < bi))
        bm = jnp.where(better, m, bm)
        bi = jnp.where(better, idx, bi)
    return bi


def _sc_idx_body(x_hbm, idx_hbm, row_a, row_b, idx_v, sem_a, sem_b):
    wid = lax.axis_index("s") * NC + lax.axis_index("c")
    r0 = wid * ROWS_PER_W

    cp_a = pltpu.async_copy(x_hbm.at[r0], row_a, sem_a)
    cp_b = pltpu.async_copy(x_hbm.at[r0 + 1], row_b, sem_b)

    cp_a.wait()
    idx_a = _row_argmax(row_a)
    cp_b.wait()
    idx_b = _row_argmax(row_b)

    lanes = lax.iota(jnp.int32, L)
    idx_v[...] = jnp.where(lanes == 0, idx_a, jnp.where(lanes == 1, idx_b, 0))
    pltpu.sync_copy(idx_v, idx_hbm.at[wid])


def _sc_argmax(probs):
    mesh = plsc.VectorSubcoreMesh(core_axis_name="c", subcore_axis_name="s")
    fn = functools.partial(
        pl.kernel,
        mesh=mesh,
        out_type=jax.ShapeDtypeStruct((NW, L), jnp.int32),
        scratch_types=[
            pltpu.VMEM((C,), jnp.float32),
            pltpu.VMEM((C,), jnp.float32),
            pltpu.VMEM((L,), jnp.int32),
            pltpu.SemaphoreType.DMA,
            pltpu.SemaphoreType.DMA,
        ],
    )(_sc_idx_body)
    return fn(probs)


def _zeros_body(o_ref):
    o_ref[...] = jnp.zeros_like(o_ref)


def _make_zeros():
    return pl.pallas_call(
        _zeros_body,
        grid=(C // ZBLK,),
        out_specs=pl.BlockSpec((R, ZBLK), lambda j: (0, j)),
        out_shape=jax.ShapeDtypeStruct((R, C), jnp.float32),
    )()


def _flat_idx(idx_ref, r):
    return idx_ref[r // ROWS_PER_W, r % ROWS_PER_W]


def _patch_body(idx_ref, z_ref, o_ref):
    g = pl.program_id(0)
    i = pl.program_id(1)
    base = (_flat_idx(idx_ref, g * 8 + i) // PBLK) * PBLK
    cols = lax.broadcasted_iota(jnp.int32, (1, PBLK), 1) + base
    rows = [
        jnp.where(cols == _flat_idx(idx_ref, g * 8 + k), 1.0, 0.0).astype(
            jnp.float32
        )
        for k in range(8)
    ]
    o_ref[...] = jnp.concatenate(rows, axis=0)


def _patch(zeros, idx):
    grid_spec = pltpu.PrefetchScalarGridSpec(
        num_scalar_prefetch=1,
        grid=(R // 8, 8),
        in_specs=[
            pl.BlockSpec(
                (8, PBLK),
                lambda g, i, idx_ref: (g, _flat_idx(idx_ref, g * 8 + i) // PBLK),
            )
        ],
        out_specs=pl.BlockSpec(
            (8, PBLK),
            lambda g, i, idx_ref: (g, _flat_idx(idx_ref, g * 8 + i) // PBLK),
        ),
    )
    return pl.pallas_call(
        _patch_body,
        grid_spec=grid_spec,
        out_shape=jax.ShapeDtypeStruct((R, C), jnp.float32),
        input_output_aliases={1: 0},
    )(idx, zeros)


def kernel(probs):
    idx = _sc_argmax(probs)
    zeros = _make_zeros()
    return _patch(zeros, idx)


# speedup vs baseline: 1.2589x; 1.2589x over previous
import functools

import jax
import jax.numpy as jnp
from jax import lax
from jax.experimental import pallas as pl
from jax.experimental.pallas import tpu as pltpu
from jax.experimental.pallas import tpu_sc as plsc

R, C = 64, 32768
L = 16
NC, NS = 2, 16
NW = NC * NS
ROWS_PER_W = R // NW
NCHUNK = C // L
UNROLL = 8
NITER = NCHUNK // UNROLL

ZBLK = 4096
PBLK = 128


def _row_argmax(row_v):
    lanes = lax.iota(jnp.int32, L)

    def body(i, carry):
        vmaxs, vits, iv = carry
        base = i * (UNROLL * L)
        new_maxs, new_its = [], []
        for k in range(UNROLL):
            v = row_v[pl.ds(base + k * L, L)]
            m = v > vmaxs[k]
            new_maxs.append(jnp.where(m, v, vmaxs[k]))
            new_its.append(jnp.where(m, iv, vits[k]))
        return tuple(new_maxs), tuple(new_its), iv + 1

    vmax0 = tuple(jnp.full((L,), -jnp.inf, jnp.float32) for _ in range(UNROLL))
    vit0 = tuple(jnp.zeros((L,), jnp.int32) for _ in range(UNROLL))
    iv0 = jnp.zeros((L,), jnp.int32)
    vmaxs, vits, _ = lax.fori_loop(0, NITER, body, (vmax0, vit0, iv0))

    vmaxs = list(vmaxs)
    vflat = [(vits[k] * UNROLL + k) * L + lanes for k in range(UNROLL)]

    n = UNROLL
    while n > 1:
        h = n // 2
        for k in range(h):
            a_m, a_i = vmaxs[k], vflat[k]
            b_m, b_i = vmaxs[k + h], vflat[k + h]
            better = (b_m > a_m) | ((b_m == a_m) & (b_i < a_i))
            vmaxs[k] = jnp.where(better, b_m, a_m)
            vflat[k] = jnp.where(better, b_i, a_i)
        n = h
    vmax, vidx = vmaxs[0], vflat[0]

    bm, bi = vmax[0], vidx[0]
    for i in range(1, L):
        m, idx = vmax[i], vidx[i]
        better = (m > bm) | ((m == bm) & (idx < bi))
        bm = jnp.where(better, m, bm)
        bi = jnp.where(better, idx, bi)
    return bi


def _sc_idx_body(x_hbm, idx_hbm, row_a, row_b, idx_v, sem_a, sem_b):
    wid = lax.axis_index("s") * NC + lax.axis_index("c")
    r0 = wid * ROWS_PER_W

    cp_a = pltpu.async_copy(x_hbm.at[r0], row_a, sem_a)
    cp_b = pltpu.async_copy(x_hbm.at[r0 + 1], row_b, sem_b)

    cp_a.wait()
    idx_a = _row_argmax(row_a)
    cp_b.wait()
    idx_b = _row_argmax(row_b)

    lanes = lax.iota(jnp.int32, L)
    idx_v[...] = jnp.where(lanes == 0, idx_a, jnp.where(lanes == 1, idx_b, 0))
    pltpu.sync_copy(idx_v, idx_hbm.at[wid])


def _sc_argmax(probs):
    mesh = plsc.VectorSubcoreMesh(core_axis_name="c", subcore_axis_name="s")
    fn = functools.partial(
        pl.kernel,
        mesh=mesh,
        out_type=jax.ShapeDtypeStruct((NW, L), jnp.int32),
        scratch_types=[
            pltpu.VMEM((C,), jnp.float32),
            pltpu.VMEM((C,), jnp.float32),
            pltpu.VMEM((L,), jnp.int32),
            pltpu.SemaphoreType.DMA,
            pltpu.SemaphoreType.DMA,
        ],
    )(_sc_idx_body)
    return fn(probs)


def _zeros_body(o_ref):
    o_ref[...] = jnp.zeros_like(o_ref)


def _make_zeros():
    return pl.pallas_call(
        _zeros_body,
        grid=(C // ZBLK,),
        out_specs=pl.BlockSpec((R, ZBLK), lambda j: (0, j)),
        out_shape=jax.ShapeDtypeStruct((R, C), jnp.float32),
    )()


def _flat_idx(idx_ref, r):
    return idx_ref[r // ROWS_PER_W, r % ROWS_PER_W]


def _patch_body(idx_ref, z_ref, o_ref):
    g = pl.program_id(0)
    i = pl.program_id(1)
    base = (_flat_idx(idx_ref, g * 8 + i) // PBLK) * PBLK
    cols = lax.broadcasted_iota(jnp.int32, (1, PBLK), 1) + base
    rows = [
        jnp.where(cols == _flat_idx(idx_ref, g * 8 + k), 1.0, 0.0).astype(
            jnp.float32
        )
        for k in range(8)
    ]
    o_ref[...] = jnp.concatenate(rows, axis=0)


def _patch(zeros, idx):
    grid_spec = pltpu.PrefetchScalarGridSpec(
        num_scalar_prefetch=1,
        grid=(R // 8, 8),
        in_specs=[pl.BlockSpec(memory_space=pl.ANY)],
        out_specs=pl.BlockSpec(
            (8, PBLK),
            lambda g, i, idx_ref: (g, _flat_idx(idx_ref, g * 8 + i) // PBLK),
        ),
    )
    return pl.pallas_call(
        _patch_body,
        grid_spec=grid_spec,
        out_shape=jax.ShapeDtypeStruct((R, C), jnp.float32),
        input_output_aliases={1: 0},
    )(idx, zeros)


def kernel(probs):
    idx = _sc_argmax(probs)
    zeros = _make_zeros()
    return _patch(zeros, idx)


# speedup vs baseline: 2.0242x; 1.6079x over previous
import functools

import jax
import jax.numpy as jnp
from jax import lax
from jax.experimental import pallas as pl
from jax.experimental.pallas import tpu as pltpu
from jax.experimental.pallas import tpu_sc as plsc

R, C = 64, 32768
L = 16
NC, NS = 2, 16
NW = NC * NS
ROWS_PER_W = R // NW
NCHUNK = C // L
UNROLL = 8
NITER = NCHUNK // UNROLL

ZBLK = 4096
PBLK = 128


def _row_argmax(row_v):
    lanes = lax.iota(jnp.int32, L)

    def body(i, carry):
        vmaxs, vits, iv = carry
        base = i * (UNROLL * L)
        new_maxs, new_its = [], []
        for k in range(UNROLL):
            v = row_v[pl.ds(base + k * L, L)]
            m = v > vmaxs[k]
            new_maxs.append(jnp.where(m, v, vmaxs[k]))
            new_its.append(jnp.where(m, iv, vits[k]))
        return tuple(new_maxs), tuple(new_its), iv + 1

    vmax0 = tuple(jnp.full((L,), -jnp.inf, jnp.float32) for _ in range(UNROLL))
    vit0 = tuple(jnp.zeros((L,), jnp.int32) for _ in range(UNROLL))
    iv0 = jnp.zeros((L,), jnp.int32)
    vmaxs, vits, _ = lax.fori_loop(0, NITER, body, (vmax0, vit0, iv0))

    vmaxs = list(vmaxs)
    vflat = [(vits[k] * UNROLL + k) * L + lanes for k in range(UNROLL)]

    n = UNROLL
    while n > 1:
        h = n // 2
        for k in range(h):
            a_m, a_i = vmaxs[k], vflat[k]
            b_m, b_i = vmaxs[k + h], vflat[k + h]
            better = (b_m > a_m) | ((b_m == a_m) & (b_i < a_i))
            vmaxs[k] = jnp.where(better, b_m, a_m)
            vflat[k] = jnp.where(better, b_i, a_i)
        n = h
    vmax, vidx = vmaxs[0], vflat[0]

    bm, bi = vmax[0], vidx[0]
    for i in range(1, L):
        m, idx = vmax[i], vidx[i]
        better = (m > bm) | ((m == bm) & (idx < bi))
        bm = jnp.where(better, m, bm)
        bi = jnp.where(better, idx, bi)
    return bi


def _sc_idx_body(x_hbm, idx_hbm, row_a, row_b, idx_v, sem_a, sem_b):
    wid = lax.axis_index("s") * NC + lax.axis_index("c")
    r0 = wid * ROWS_PER_W

    cp_a = pltpu.async_copy(x_hbm.at[r0], row_a, sem_a)
    cp_b = pltpu.async_copy(x_hbm.at[r0 + 1], row_b, sem_b)

    cp_a.wait()
    idx_a = _row_argmax(row_a)
    cp_b.wait()
    idx_b = _row_argmax(row_b)

    idx_v[...] = jnp.full((L,), idx_a, jnp.int32)
    pltpu.sync_copy(idx_v, idx_hbm.at[r0])
    idx_v[...] = jnp.full((L,), idx_b, jnp.int32)
    pltpu.sync_copy(idx_v, idx_hbm.at[r0 + 1])


def _sc_argmax(probs):
    mesh = plsc.VectorSubcoreMesh(core_axis_name="c", subcore_axis_name="s")
    fn = functools.partial(
        pl.kernel,
        mesh=mesh,
        out_type=jax.ShapeDtypeStruct((R, L), jnp.int32),
        scratch_types=[
            pltpu.VMEM((C,), jnp.float32),
            pltpu.VMEM((C,), jnp.float32),
            pltpu.VMEM((L,), jnp.int32),
            pltpu.SemaphoreType.DMA,
            pltpu.SemaphoreType.DMA,
        ],
    )(_sc_idx_body)
    return fn(probs)


def _zeros_body(o_ref):
    o_ref[...] = jnp.zeros_like(o_ref)


def _make_zeros():
    return pl.pallas_call(
        _zeros_body,
        grid=(C // ZBLK,),
        out_specs=pl.BlockSpec((R, ZBLK), lambda j: (0, j)),
        out_shape=jax.ShapeDtypeStruct((R, C), jnp.float32),
    )()


def _patch_body(idx_smem, idx_vmem, z_any, o_any, seg_v, sem):
    idxc = idx_vmem[:, 0:1]
    lane = idxc % PBLK
    cols = lax.broadcasted_iota(jnp.int32, (R, PBLK), 1)
    seg_v[...] = jnp.where(cols == lane, 1.0, 0.0).astype(jnp.float32)
    copies = []
    for r in range(R):
        base = (idx_smem[r, 0] // PBLK) * PBLK
        cp = pltpu.make_async_copy(
            seg_v.at[r], o_any.at[r, pl.ds(base, PBLK)], sem
        )
        cp.start()
        copies.append(cp)
    for cp in copies:
        cp.wait()


def _patch(zeros, idx):
    return pl.pallas_call(
        _patch_body,
        in_specs=[
            pl.BlockSpec(memory_space=pltpu.MemorySpace.SMEM),
            pl.BlockSpec(memory_space=pltpu.MemorySpace.VMEM),
            pl.BlockSpec(memory_space=pl.ANY),
        ],
        out_specs=pl.BlockSpec(memory_space=pl.ANY),
        out_shape=jax.ShapeDtypeStruct((R, C), jnp.float32),
        scratch_shapes=[
            pltpu.VMEM((R, PBLK), jnp.float32),
            pltpu.SemaphoreType.DMA,
        ],
        input_output_aliases={2: 0},
    )(idx, idx, zeros)


def kernel(probs):
    idx = _sc_argmax(probs)
    zeros = _make_zeros()
    return _patch(zeros, idx)


# speedup vs baseline: 2.0377x; 1.0067x over previous
import functools

import jax
import jax.numpy as jnp
from jax import lax
from jax.experimental import pallas as pl
from jax.experimental.pallas import tpu as pltpu
from jax.experimental.pallas import tpu_sc as plsc

R, C = 64, 32768
RH = R // 2
L = 16
NC, NS = 2, 16
NW = NC * NS
NCHUNK = C // L
UNROLL = 8
NITER = NCHUNK // UNROLL

TBLK = 4096
PBLK = 128


def _row_argmax(row_v):
    lanes = lax.iota(jnp.int32, L)

    def body(i, carry):
        vmaxs, vits, iv = carry
        base = i * (UNROLL * L)
        new_maxs, new_its = [], []
        for k in range(UNROLL):
            v = row_v[pl.ds(base + k * L, L)]
            m = v > vmaxs[k]
            new_maxs.append(jnp.where(m, v, vmaxs[k]))
            new_its.append(jnp.where(m, iv, vits[k]))
        return tuple(new_maxs), tuple(new_its), iv + 1

    vmax0 = tuple(jnp.full((L,), -jnp.inf, jnp.float32) for _ in range(UNROLL))
    vit0 = tuple(jnp.zeros((L,), jnp.int32) for _ in range(UNROLL))
    iv0 = jnp.zeros((L,), jnp.int32)
    vmaxs, vits, _ = lax.fori_loop(0, NITER, body, (vmax0, vit0, iv0))

    vmaxs = list(vmaxs)
    vflat = [(vits[k] * UNROLL + k) * L + lanes for k in range(UNROLL)]

    n = UNROLL
    while n > 1:
        h = n // 2
        for k in range(h):
            a_m, a_i = vmaxs[k], vflat[k]
            b_m, b_i = vmaxs[k + h], vflat[k + h]
            better = (b_m > a_m) | ((b_m == a_m) & (b_i < a_i))
            vmaxs[k] = jnp.where(better, b_m, a_m)
            vflat[k] = jnp.where(better, b_i, a_i)
        n = h
    vmax, vidx = vmaxs[0], vflat[0]

    bm, bi = vmax[0], vidx[0]
    for i in range(1, L):
        m, idx = vmax[i], vidx[i]
        better = (m > bm) | ((m == bm) & (idx < bi))
        bm = jnp.where(better, m, bm)
        bi = jnp.where(better, idx, bi)
    return bi


def _sc_idx_body(x_hbm, idx_hbm, row_a, idx_v, sem_a):
    wid = lax.axis_index("s") * NC + lax.axis_index("c")

    cp_a = pltpu.async_copy(x_hbm.at[wid], row_a, sem_a)
    cp_a.wait()
    idx_a = _row_argmax(row_a)
    idx_v[...] = jnp.full((L,), idx_a, jnp.int32)
    pltpu.sync_copy(idx_v, idx_hbm.at[wid])


def _sc_argmax(probs_top):
    mesh = plsc.VectorSubcoreMesh(core_axis_name="c", subcore_axis_name="s")
    fn = functools.partial(
        pl.kernel,
        mesh=mesh,
        out_type=jax.ShapeDtypeStruct((RH, L), jnp.int32),
        scratch_types=[
            pltpu.VMEM((C,), jnp.float32),
            pltpu.VMEM((L,), jnp.int32),
            pltpu.SemaphoreType.DMA,
        ],
    )(_sc_idx_body)
    return fn(probs_top)


def _zoh_body(x_ref, o_ref, max_s, idx_s):
    p = pl.program_id(0)
    j = pl.program_id(1)

    @pl.when(p == 0)
    def _phase0():
        @pl.when(j == 0)
        def _init():
            max_s[...] = jnp.full((RH, 1), -jnp.inf, jnp.float32)
            idx_s[...] = jnp.zeros((RH, 1), jnp.int32)

        x = x_ref[...]
        bm = jnp.max(x, axis=1, keepdims=True)
        bi = jnp.argmax(x, axis=1).astype(jnp.int32)[:, None] + j * TBLK
        upd = bm > max_s[...]
        idx_s[...] = jnp.where(upd, bi, idx_s[...])
        max_s[...] = jnp.where(upd, bm, max_s[...])

    @pl.when(p == 1)
    def _phase1():
        cols = lax.broadcasted_iota(jnp.int32, (RH, TBLK), 1) + j * TBLK
        bot = jnp.where(cols == idx_s[...], 1.0, 0.0).astype(jnp.float32)
        o_ref[...] = jnp.concatenate(
            [jnp.zeros((RH, TBLK), jnp.float32), bot], axis=0
        )


def _zeros_onehot(probs):
    return pl.pallas_call(
        _zoh_body,
        grid=(2, C // TBLK),
        in_specs=[pl.BlockSpec((RH, TBLK), lambda p, j: (1, j * (1 - p)))],
        out_specs=pl.BlockSpec((R, TBLK), lambda p, j: (0, j * p)),
        out_shape=jax.ShapeDtypeStruct((R, C), jnp.float32),
        scratch_shapes=[
            pltpu.VMEM((RH, 1), jnp.float32),
            pltpu.VMEM((RH, 1), jnp.int32),
        ],
    )(probs)


def _patch_body(idx_smem, idx_vmem, z_any, o_any, seg_v, sem):
    idxc = idx_vmem[:, 0:1]
    lane = idxc % PBLK
    cols = lax.broadcasted_iota(jnp.int32, (RH, PBLK), 1)
    seg_v[...] = jnp.where(cols == lane, 1.0, 0.0).astype(jnp.float32)
    copies = []
    for r in range(RH):
        base = (idx_smem[r, 0] // PBLK) * PBLK
        cp = pltpu.make_async_copy(
            seg_v.at[r], o_any.at[r, pl.ds(base, PBLK)], sem
        )
        cp.start()
        copies.append(cp)
    for cp in copies:
        cp.wait()


def _patch(buf, idx):
    return pl.pallas_call(
        _patch_body,
        in_specs=[
            pl.BlockSpec(memory_space=pltpu.MemorySpace.SMEM),
            pl.BlockSpec(memory_space=pltpu.MemorySpace.VMEM),
            pl.BlockSpec(memory_space=pl.ANY),
        ],
        out_specs=pl.BlockSpec(memory_space=pl.ANY),
        out_shape=jax.ShapeDtypeStruct((R, C), jnp.float32),
        scratch_shapes=[
            pltpu.VMEM((RH, PBLK), jnp.float32),
            pltpu.SemaphoreType.DMA,
        ],
        input_output_aliases={2: 0},
    )(idx, idx, buf)


def kernel(probs):
    idx = _sc_argmax(probs)
    buf = _zeros_onehot(probs)
    return _patch(buf, idx)


# speedup vs baseline: 2.0682x; 1.0150x over previous
import functools

import jax
import jax.numpy as jnp
from jax import lax
from jax.experimental import pallas as pl
from jax.experimental.pallas import tpu as pltpu
from jax.experimental.pallas import tpu_sc as plsc

R, C = 64, 32768
L = 16
NC, NS = 2, 16
NW = NC * NS
ROWS_PER_W = R // NW
NCHUNK = C // L
UNROLL = 8
NITER = NCHUNK // UNROLL

ZSL = C // NS
SEGW = 128


def _row_argmax(row_v):
    lanes = lax.iota(jnp.int32, L)

    def body(i, carry):
        vmaxs, vits, iv = carry
        base = i * (UNROLL * L)
        new_maxs, new_its = [], []
        for k in range(UNROLL):
            v = row_v[pl.ds(base + k * L, L)]
            m = v > vmaxs[k]
            new_maxs.append(jnp.where(m, v, vmaxs[k]))
            new_its.append(jnp.where(m, iv, vits[k]))
        return tuple(new_maxs), tuple(new_its), iv + 1

    vmax0 = tuple(jnp.full((L,), -jnp.inf, jnp.float32) for _ in range(UNROLL))
    vit0 = tuple(jnp.zeros((L,), jnp.int32) for _ in range(UNROLL))
    iv0 = jnp.zeros((L,), jnp.int32)
    vmaxs, vits, _ = lax.fori_loop(0, NITER, body, (vmax0, vit0, iv0))

    vmaxs = list(vmaxs)
    vflat = [(vits[k] * UNROLL + k) * L + lanes for k in range(UNROLL)]

    n = UNROLL
    while n > 1:
        h = n // 2
        for k in range(h):
            a_m, a_i = vmaxs[k], vflat[k]
            b_m, b_i = vmaxs[k + h], vflat[k + h]
            better = (b_m > a_m) | ((b_m == a_m) & (b_i < a_i))
            vmaxs[k] = jnp.where(better, b_m, a_m)
            vflat[k] = jnp.where(better, b_i, a_i)
        n = h
    vmax, vidx = vmaxs[0], vflat[0]

    bm, bi = vmax[0], vidx[0]
    for i in range(1, L):
        m, idx = vmax[i], vidx[i]
        better = (m > bm) | ((m == bm) & (idx < bi))
        bm = jnp.where(better, m, bm)
        bi = jnp.where(better, idx, bi)
    return bi


def _fill_patch(patch_v, idx):
    lanes = lax.iota(jnp.int32, L)
    lane = idx % SEGW
    for k in range(SEGW // L):
        patch_v[pl.ds(k * L, L)] = jnp.where(
            lanes + k * L == lane, jnp.float32(1.0), jnp.float32(0.0)
        )


def _sc_body(
    x_hbm, out_hbm,
    row_a, row_b, zstage, patch_a, patch_b, zbuf,
    sem_a, sem_b, sem_zs, sem_za, sem_zb, sem_p,
):
    sid = lax.axis_index("s")
    wid = sid * NC + lax.axis_index("c")
    r0 = wid * ROWS_PER_W

    cp_a = pltpu.async_copy(x_hbm.at[r0], row_a, sem_a)
    cp_b = pltpu.async_copy(x_hbm.at[r0 + 1], row_b, sem_b)

    zero = jnp.zeros((L,), jnp.float32)
    for k in range(ZSL // L):
        zstage[pl.ds(k * L, L)] = zero
    pltpu.async_copy(zstage, zbuf.at[pl.ds(sid * ZSL, ZSL)], sem_zs).wait()
    plsc.subcore_barrier()

    cp_za = pltpu.async_copy(zbuf, out_hbm.at[r0], sem_za)
    cp_zb = pltpu.async_copy(zbuf, out_hbm.at[r0 + 1], sem_zb)

    cp_a.wait()
    idx_a = _row_argmax(row_a)
    _fill_patch(patch_a, idx_a)
    base_a = (idx_a // SEGW) * SEGW
    cp_za.wait()
    cp_pa = pltpu.async_copy(
        patch_a, out_hbm.at[r0, pl.ds(base_a, SEGW)], sem_p
    )

    cp_b.wait()
    idx_b = _row_argmax(row_b)
    _fill_patch(patch_b, idx_b)
    base_b = (idx_b // SEGW) * SEGW
    cp_zb.wait()
    cp_pb = pltpu.async_copy(
        patch_b, out_hbm.at[r0 + 1, pl.ds(base_b, SEGW)], sem_p
    )

    cp_pa.wait()
    cp_pb.wait()


def kernel(probs):
    mesh = plsc.VectorSubcoreMesh(core_axis_name="c", subcore_axis_name="s")
    sc_fn = functools.partial(
        pl.kernel,
        mesh=mesh,
        out_type=jax.ShapeDtypeStruct((R, C), jnp.float32),
        scratch_types=[
            pltpu.VMEM((C,), jnp.float32),
            pltpu.VMEM((C,), jnp.float32),
            pltpu.VMEM((ZSL,), jnp.float32),
            pltpu.VMEM((SEGW,), jnp.float32),
            pltpu.VMEM((SEGW,), jnp.float32),
            pltpu.VMEM_SHARED((C,), jnp.float32),
            pltpu.SemaphoreType.DMA,
            pltpu.SemaphoreType.DMA,
            pltpu.SemaphoreType.DMA,
            pltpu.SemaphoreType.DMA,
            pltpu.SemaphoreType.DMA,
            pltpu.SemaphoreType.DMA,
        ],
    )(_sc_body)
    return sc_fn(probs)


# speedup vs baseline: 2.1561x; 1.0425x over previous
import functools

import jax
import jax.numpy as jnp
from jax import lax
from jax.experimental import pallas as pl
from jax.experimental.pallas import tpu as pltpu
from jax.experimental.pallas import tpu_sc as plsc

R, C = 64, 32768
L = 16
NC, NS = 2, 16
NW = NC * NS
ROWS_PER_W = R // NW
NCHUNK = C // L
UNROLL = 8
NITER = NCHUNK // UNROLL


def _row_argmax(row_v):
    lanes = lax.iota(jnp.int32, L)

    def body(i, carry):
        vmaxs, vits, iv = carry
        base = i * (UNROLL * L)
        new_maxs, new_its = [], []
        for k in range(UNROLL):
            v = row_v[pl.ds(base + k * L, L)]
            m = v > vmaxs[k]
            new_maxs.append(jnp.where(m, v, vmaxs[k]))
            new_its.append(jnp.where(m, iv, vits[k]))
        return tuple(new_maxs), tuple(new_its), iv + 1

    vmax0 = tuple(jnp.full((L,), -jnp.inf, jnp.float32) for _ in range(UNROLL))
    vit0 = tuple(jnp.zeros((L,), jnp.int32) for _ in range(UNROLL))
    iv0 = jnp.zeros((L,), jnp.int32)
    vmaxs, vits, _ = lax.fori_loop(0, NITER, body, (vmax0, vit0, iv0))

    vmaxs = list(vmaxs)
    vflat = [
        (vits[k] * UNROLL + k) * L + lanes for k in range(UNROLL)
    ]

    n = UNROLL
    while n > 1:
        h = n // 2
        for k in range(h):
            a_m, a_i = vmaxs[k], vflat[k]
            b_m, b_i = vmaxs[k + h], vflat[k + h]
            better = (b_m > a_m) | ((b_m == a_m) & (b_i < a_i))
            vmaxs[k] = jnp.where(better, b_m, a_m)
            vflat[k] = jnp.where(better, b_i, a_i)
        n = h
    vmax, vidx = vmaxs[0], vflat[0]

    bm, bi = vmax[0], vidx[0]
    for i in range(1, L):
        m, idx = vmax[i], vidx[i]
        better = (m > bm) | ((m == bm) & (idx < bi))
        bm = jnp.where(better, m, bm)
        bi = jnp.where(better, idx, bi)
    return bi


def _set_at(out_v, idx, val):
    lanes = lax.iota(jnp.int32, L)
    base = (idx // L) * L
    lane = idx - base
    vec = jnp.where(lanes == lane, jnp.float32(val), jnp.float32(0.0))
    out_v[pl.ds(base, L)] = vec


def _sc_body(x_hbm, out_hbm, row_a, row_b, out_v, sem_a, sem_b, sem_o):
    wid = lax.axis_index("s") * NC + lax.axis_index("c")
    r0 = wid * ROWS_PER_W

    cp_a = pltpu.async_copy(x_hbm.at[r0], row_a, sem_a)
    cp_b = pltpu.async_copy(x_hbm.at[r0 + 1], row_b, sem_b)

    zero = jnp.zeros((L,), jnp.float32)

    def zbody(i, _):
        for k in range(UNROLL):
            out_v[pl.ds((i * UNROLL + k) * L, L)] = zero
        return 0

    lax.fori_loop(0, NITER, zbody, 0)

    cp_a.wait()
    idx_a = _row_argmax(row_a)
    _set_at(out_v, idx_a, 1.0)
    cp_oa = pltpu.async_copy(out_v, out_hbm.at[r0], sem_o)

    cp_b.wait()
    idx_b = _row_argmax(row_b)
    cp_oa.wait()
    _set_at(out_v, idx_a, 0.0)
    _set_at(out_v, idx_b, 1.0)
    pltpu.sync_copy(out_v, out_hbm.at[r0 + 1])


def kernel(probs):
    mesh = plsc.VectorSubcoreMesh(core_axis_name="c", subcore_axis_name="s")
    sc_fn = functools.partial(
        pl.kernel,
        mesh=mesh,
        out_type=jax.ShapeDtypeStruct((R, C), jnp.float32),
        scratch_types=[
            pltpu.VMEM((C,), jnp.float32),
            pltpu.VMEM((C,), jnp.float32),
            pltpu.VMEM((C,), jnp.float32),
            pltpu.SemaphoreType.DMA,
            pltpu.SemaphoreType.DMA,
            pltpu.SemaphoreType.DMA,
        ],
    )(_sc_body)
    return sc_fn(probs)


# speedup vs baseline: 2.1616x; 1.0026x over previous
import functools

import jax
import jax.numpy as jnp
from jax import lax
from jax.experimental import pallas as pl
from jax.experimental.pallas import tpu as pltpu
from jax.experimental.pallas import tpu_sc as plsc

R, C = 64, 32768
L = 16
NC, NS = 2, 16
NW = NC * NS
ROWS_PER_W = R // NW
NCHUNK = C // L
UNROLL = 16
NITER = NCHUNK // UNROLL


def _row_argmax(row_v):
    lanes = lax.iota(jnp.int32, L)

    def body(i, carry):
        vmaxs, vits, iv = carry
        base = i * (UNROLL * L)
        new_maxs, new_its = [], []
        for k in range(UNROLL):
            v = row_v[pl.ds(base + k * L, L)]
            m = v > vmaxs[k]
            new_maxs.append(jnp.where(m, v, vmaxs[k]))
            new_its.append(jnp.where(m, iv, vits[k]))
        return tuple(new_maxs), tuple(new_its), iv + 1

    vmax0 = tuple(jnp.full((L,), -jnp.inf, jnp.float32) for _ in range(UNROLL))
    vit0 = tuple(jnp.zeros((L,), jnp.int32) for _ in range(UNROLL))
    iv0 = jnp.zeros((L,), jnp.int32)
    vmaxs, vits, _ = lax.fori_loop(0, NITER, body, (vmax0, vit0, iv0))

    vmaxs = list(vmaxs)
    vflat = [
        (vits[k] * UNROLL + k) * L + lanes for k in range(UNROLL)
    ]

    n = UNROLL
    while n > 1:
        h = n // 2
        for k in range(h):
            a_m, a_i = vmaxs[k], vflat[k]
            b_m, b_i = vmaxs[k + h], vflat[k + h]
            better = (b_m > a_m) | ((b_m == a_m) & (b_i < a_i))
            vmaxs[k] = jnp.where(better, b_m, a_m)
            vflat[k] = jnp.where(better, b_i, a_i)
        n = h
    vmax, vidx = vmaxs[0], vflat[0]

    bm, bi = vmax[0], vidx[0]
    for i in range(1, L):
        m, idx = vmax[i], vidx[i]
        better = (m > bm) | ((m == bm) & (idx < bi))
        bm = jnp.where(better, m, bm)
        bi = jnp.where(better, idx, bi)
    return bi


def _set_at(out_v, idx, val):
    lanes = lax.iota(jnp.int32, L)
    base = (idx // L) * L
    lane = idx - base
    vec = jnp.where(lanes == lane, jnp.float32(val), jnp.float32(0.0))
    out_v[pl.ds(base, L)] = vec


def _sc_body(x_hbm, out_hbm, row_a, row_b, out_v, sem_a, sem_b, sem_o):
    wid = lax.axis_index("s") * NC + lax.axis_index("c")
    r0 = wid * ROWS_PER_W

    cp_a = pltpu.async_copy(x_hbm.at[r0], row_a, sem_a)
    cp_b = pltpu.async_copy(x_hbm.at[r0 + 1], row_b, sem_b)

    zero = jnp.zeros((L,), jnp.float32)

    def zbody(i, _):
        for k in range(UNROLL):
            out_v[pl.ds((i * UNROLL + k) * L, L)] = zero
        return 0

    lax.fori_loop(0, NITER, zbody, 0)

    cp_a.wait()
    idx_a = _row_argmax(row_a)
    _set_at(out_v, idx_a, 1.0)
    cp_oa = pltpu.async_copy(out_v, out_hbm.at[r0], sem_o)

    cp_b.wait()
    idx_b = _row_argmax(row_b)
    cp_oa.wait()
    _set_at(out_v, idx_a, 0.0)
    _set_at(out_v, idx_b, 1.0)
    pltpu.sync_copy(out_v, out_hbm.at[r0 + 1])


def kernel(probs):
    mesh = plsc.VectorSubcoreMesh(core_axis_name="c", subcore_axis_name="s")
    sc_fn = functools.partial(
        pl.kernel,
        mesh=mesh,
        out_type=jax.ShapeDtypeStruct((R, C), jnp.float32),
        scratch_types=[
            pltpu.VMEM((C,), jnp.float32),
            pltpu.VMEM((C,), jnp.float32),
            pltpu.VMEM((C,), jnp.float32),
            pltpu.SemaphoreType.DMA,
            pltpu.SemaphoreType.DMA,
            pltpu.SemaphoreType.DMA,
        ],
    )(_sc_body)
    return sc_fn(probs)
